# baseline (speedup 1.0000x reference)
import jax
import jax.numpy as jnp
from jax.experimental import pallas as pl
from jax.experimental.pallas import tpu as pltpu

_CHUNK = 2500


def _outer(x_any, t_any, w_ref, b_ref, o_any):
    w = w_ref[...]
    bb = b_ref[...]

    def _inner(x_blk, t_blk, o_blk):
        x = x_blk[...]
        y = jax.lax.dot_general(
            x, w, (((1,), (1,)), ((), ())),
            preferred_element_type=jnp.float32,
        )
        m = jnp.transpose(t_blk[...].reshape(1, _CHUNK))
        o_blk[...] = jnp.where(m == 1, y + bb, x)

    n, d = x_any.shape
    pipeline = pltpu.emit_pipeline(
        _inner,
        grid=(n // _CHUNK,),
        in_specs=[
            pl.BlockSpec((_CHUNK, d), lambda i: (i, 0)),
            pl.BlockSpec((1, 1, _CHUNK), lambda i: (i, 0, 0)),
        ],
        out_specs=[pl.BlockSpec((_CHUNK, d), lambda i: (i, 0))],
    )
    pipeline(x_any, t_any, o_any)


def kernel(node_emb, node_type, W, b):
    n, d = node_emb.shape
    t = node_type.astype(jnp.int8).reshape(n // _CHUNK, 1, _CHUNK)
    b2 = b.reshape(1, d)
    return pl.pallas_call(
        _outer,
        grid=(1,),
        in_specs=[
            pl.BlockSpec(memory_space=pl.ANY),
            pl.BlockSpec(memory_space=pl.ANY),
            pl.BlockSpec((d, d), lambda i: (0, 0)),
            pl.BlockSpec((1, d), lambda i: (0, 0)),
        ],
        out_specs=pl.BlockSpec(memory_space=pl.ANY),
        out_shape=jax.ShapeDtypeStruct((n, d), jnp.float32),
    )(node_emb, t, W, b2)


# speedup vs baseline: 1.4717x; 1.4717x over previous
import jax
import jax.numpy as jnp
from jax.experimental import pallas as pl

_BLOCK = 20000


def _fused_update(x_ref, t_ref, w_ref, b_ref, o_ref):
    x = x_ref[...]
    y = jax.lax.dot_general(
        x, w_ref[...], (((1,), (1,)), ((), ())),
        preferred_element_type=jnp.float32,
    )
    m = jnp.transpose(t_ref[...].reshape(1, _BLOCK))
    o_ref[...] = jnp.where(m == 1, y + b_ref[...], x)


def kernel(node_emb, node_type, W, b):
    n, d = node_emb.shape
    grid = n // _BLOCK
    t = node_type.astype(jnp.int8).reshape(grid, 1, _BLOCK)
    b2 = b.reshape(1, d)
    return pl.pallas_call(
        _fused_update,
        grid=(grid,),
        in_specs=[
            pl.BlockSpec((_BLOCK, d), lambda i: (i, 0)),
            pl.BlockSpec((1, 1, _BLOCK), lambda i: (i, 0, 0)),
            pl.BlockSpec((d, d), lambda i: (0, 0)),
            pl.BlockSpec((1, d), lambda i: (0, 0)),
        ],
        out_specs=pl.BlockSpec((_BLOCK, d), lambda i: (i, 0)),
        out_shape=jax.ShapeDtypeStruct((n, d), jnp.float32),
    )(node_emb, t, W, b2)
